# baseline (speedup 1.0000x reference)
import functools
import jax
import jax.numpy as jnp
from jax import lax
from jax.experimental import pallas as pl
from jax.experimental.pallas import tpu as pltpu
from jax.experimental.pallas import tpu_sc as plsc

S = 64.0
M = 0.4

_NC = 2
_NS = 16
_NW = _NC * _NS
_L = 16

_CH = 10000
_NB = 4


def _sc_fn(B, C):
    rows_pw = B // _NW
    cpr = C // _CH
    nchunks = rows_pw * cpr
    ngroups = nchunks // _NB
    wsize = rows_pw * C
    mesh = plsc.VectorSubcoreMesh(core_axis_name="c", subcore_axis_name="s")

    @functools.partial(
        pl.kernel,
        out_type=jax.ShapeDtypeStruct((B * C,), jnp.float32),
        mesh=mesh,
        compiler_params=pltpu.CompilerParams(needs_layout_passes=False),
        scratch_types=(
            [pltpu.VMEM((_CH,), jnp.float32) for _ in range(_NB)]
            + [pltpu.VMEM((_CH,), jnp.float32) for _ in range(_NB)]
            + [pltpu.VMEM((rows_pw,), jnp.int32)]
            + [pltpu.SemaphoreType.DMA for _ in range(2 * _NB)]
        ),
    )
    def body(cos_hbm, label_hbm, out_hbm, *scratch):
        ibufs = scratch[:_NB]
        obufs = scratch[_NB:2 * _NB]
        lab_v = scratch[2 * _NB]
        isems = scratch[2 * _NB + 1: 2 * _NB + 1 + _NB]
        osems = scratch[2 * _NB + 1 + _NB:]

        wid = lax.axis_index("s") * _NC + lax.axis_index("c")
        wbase = wid * wsize
        pltpu.sync_copy(label_hbm.at[pl.ds(wid * rows_pw, rows_pw)], lab_v)

        lanes = lax.iota(jnp.int32, _L)

        for b in range(_NB):
            pltpu.async_copy(
                cos_hbm.at[pl.ds(wbase + b * _CH, _CH)], ibufs[b], isems[b])

        @pl.loop(0, ngroups)
        def _grp(g):
            k0 = g * _NB
            for b in range(_NB):
                k = k0 + b
                off = wbase + k * _CH
                pltpu.make_async_copy(
                    cos_hbm.at[pl.ds(off, _CH)], ibufs[b], isems[b]).wait()

                @pl.when(g > 0)
                def _():
                    pltpu.make_async_copy(
                        obufs[b],
                        out_hbm.at[pl.ds(off - _NB * _CH, _CH)],
                        osems[b]).wait()

                @pl.loop(0, _CH // _L, unroll=8)
                def _vec(i):
                    sl = pl.ds(i * _L, _L)
                    obufs[b][sl] = S * ibufs[b][sl]

                row = k // cpr
                q = k - row * cpr
                vb16 = (row // _L) * _L
                lv = lab_v[pl.ds(vb16, _L)]
                lab = jnp.max(jnp.where(lanes == row - vb16, lv, -2))
                j = lab - q * _CH

                @pl.when((j >= 0) & (j < _CH))
                def _():
                    vi16 = (j // _L) * _L
                    sl = pl.ds(vi16, _L)
                    v = obufs[b][sl]
                    obufs[b][sl] = jnp.where(lanes == j - vi16, v - S * M, v)

                pltpu.async_copy(
                    obufs[b], out_hbm.at[pl.ds(off, _CH)], osems[b])

                @pl.when(g < ngroups - 1)
                def _():
                    pltpu.async_copy(
                        cos_hbm.at[pl.ds(off + _NB * _CH, _CH)],
                        ibufs[b], isems[b])

        for b in range(_NB):
            off = wbase + (nchunks - _NB + b) * _CH
            pltpu.make_async_copy(
                obufs[b], out_hbm.at[pl.ds(off, _CH)], osems[b]).wait()

    return body


def kernel(cosine, label):
    B, C = cosine.shape
    flat = _sc_fn(B, C)(cosine.reshape(B * C), label)
    return flat.reshape(B, C)


# speedup vs baseline: 2.7717x; 2.7717x over previous
import jax
import jax.numpy as jnp
from jax.experimental import pallas as pl
from jax.experimental.pallas import tpu as pltpu

S = 64.0
M = 0.4

_BLOCK_B = 1024
_BLOCK_C = 2048


def _body(lab_ref, cos_ref, out_ref):
    j = pl.program_id(1)
    lab = lab_ref[:, 0]
    col0 = j * _BLOCK_C
    cols = col0 + jax.lax.broadcasted_iota(jnp.int32, (_BLOCK_B, _BLOCK_C), 1)
    mask = (cols == lab[:, None]).astype(jnp.float32)
    out_ref[...] = S * cos_ref[...] - (S * M) * mask


def kernel(cosine, label):
    B, C = cosine.shape
    lab2d = label.reshape(B, 1)
    grid = (B // _BLOCK_B, pl.cdiv(C, _BLOCK_C))
    return pl.pallas_call(
        _body,
        grid=grid,
        in_specs=[
            pl.BlockSpec((_BLOCK_B, 1), lambda i, j: (i, 0)),
            pl.BlockSpec((_BLOCK_B, _BLOCK_C), lambda i, j: (i, j)),
        ],
        out_specs=pl.BlockSpec((_BLOCK_B, _BLOCK_C), lambda i, j: (i, j)),
        out_shape=jax.ShapeDtypeStruct((B, C), cosine.dtype),
        compiler_params=pltpu.CompilerParams(
            dimension_semantics=("parallel", "parallel"),
            vmem_limit_bytes=120 * 1024 * 1024,
        ),
    )(lab2d, cosine)


# speedup vs baseline: 10.1666x; 3.6680x over previous
import jax
import jax.numpy as jnp
from jax.experimental import pallas as pl
from jax.experimental.pallas import tpu as pltpu

S = 64.0
M = 0.4

_BLOCK_C = 2048


def _body(lab_ref, cos_ref, out_ref):
    i = pl.program_id(0)
    lab = lab_ref[:, 0]
    rows = i * _BLOCK_C + jax.lax.broadcasted_iota(
        jnp.int32, (_BLOCK_C, lab_ref.shape[0]), 0)
    mask = (rows == lab[None, :]).astype(jnp.float32)
    out_ref[...] = S * cos_ref[...] - (S * M) * mask


def kernel(cosine, label):
    B, C = cosine.shape
    cos_t = cosine.T
    out_t = pl.pallas_call(
        _body,
        grid=(pl.cdiv(C, _BLOCK_C),),
        in_specs=[
            pl.BlockSpec((B, 1), lambda i: (0, 0)),
            pl.BlockSpec((_BLOCK_C, B), lambda i: (i, 0)),
        ],
        out_specs=pl.BlockSpec((_BLOCK_C, B), lambda i: (i, 0)),
        out_shape=jax.ShapeDtypeStruct((C, B), cosine.dtype),
        compiler_params=pltpu.CompilerParams(
            dimension_semantics=("parallel",),
            vmem_limit_bytes=96 * 1024 * 1024,
        ),
    )(label.reshape(B, 1), cos_t)
    return out_t.T
